# speedup vs baseline: 5.6640x; 5.6640x over previous
import functools

import jax
import jax.numpy as jnp
from jax import lax
from jax.experimental import pallas as pl
from jax.experimental.pallas import tpu as pltpu
from jax.experimental.pallas import tpu_sc as plsc

N_NODES = 10000
N_EDGES = 320000
D_FEAT = 128

NUM_CORES = 2
NUM_SUBCORES = 16
NW = NUM_CORES * NUM_SUBCORES

CHUNK = 128
ROWS_PER_TILE = 640
N_PAD = NUM_SUBCORES * ROWS_PER_TILE

EDGES_PER_W = -(-N_EDGES // NW)
CHUNKS_PER_W = -(-EDGES_PER_W // CHUNK)
EDGES_PER_W_PAD = CHUNKS_PER_W * CHUNK
E_PAD = EDGES_PER_W_PAD * NW

ROW_BLOCK = 2000
N_BLOCKS = N_NODES // ROW_BLOCK


def _sc_agg_body(x_hbm, src_hbm, dst_hbm, zeros_hbm, agg_hbm,
                 src_v, dst_v, rows_v, agg_sh, sem):
    c = lax.axis_index("c")
    s = lax.axis_index("s")
    w = c * NUM_SUBCORES + s

    pltpu.sync_copy(src_hbm.at[w], src_v)
    pltpu.sync_copy(dst_hbm.at[w], dst_v)
    pltpu.sync_copy(zeros_hbm, agg_sh.at[pl.ds(s * ROWS_PER_TILE, ROWS_PER_TILE)])
    plsc.subcore_barrier()

    def body(j, carry):
        pltpu.async_copy(x_hbm.at[src_v.at[j]], rows_v, sem).wait()
        pltpu.sync_copy(rows_v, agg_sh.at[dst_v.at[j]], add=True)
        return carry

    lax.fori_loop(0, CHUNKS_PER_W, body, 0)
    plsc.subcore_barrier()

    pltpu.sync_copy(
        agg_sh.at[pl.ds(s * ROWS_PER_TILE, ROWS_PER_TILE)],
        agg_hbm.at[pl.ds(c * N_PAD + s * ROWS_PER_TILE, ROWS_PER_TILE)],
    )


_sc_agg = functools.partial(
    pl.kernel,
    out_type=jax.ShapeDtypeStruct((NUM_CORES * N_PAD, D_FEAT), jnp.float32),
    mesh=plsc.VectorSubcoreMesh(core_axis_name="c", subcore_axis_name="s"),
    scratch_types=[
        pltpu.VMEM((CHUNKS_PER_W, CHUNK), jnp.int32),
        pltpu.VMEM((CHUNKS_PER_W, CHUNK), jnp.int32),
        pltpu.VMEM((CHUNK, D_FEAT), jnp.float32),
        pltpu.VMEM_SHARED((N_PAD, D_FEAT), jnp.float32),
        pltpu.SemaphoreType.DMA,
    ],
)(_sc_agg_body)


def _tc_body(agg_ref, x_ref, w_ref, ws_ref, b_ref, o_ref):
    a = agg_ref[0] + agg_ref[1]
    acc = jnp.dot(a, w_ref[...], preferred_element_type=jnp.float32)
    acc = acc + jnp.dot(x_ref[...], ws_ref[...], preferred_element_type=jnp.float32)
    acc = acc + b_ref[...]
    o_ref[...] = jnp.maximum(acc, 0.0)


@jax.jit
def kernel(x, edge_index, W, W_self, b):
    src = edge_index[0]
    dst = edge_index[1]
    pad = E_PAD - N_EDGES
    src_p = jnp.concatenate([src, jnp.zeros((pad,), jnp.int32)])
    dst_p = jnp.concatenate([dst, jnp.full((pad,), N_NODES, jnp.int32)])
    src_w = src_p.reshape(NW, CHUNKS_PER_W, CHUNK)
    dst_w = dst_p.reshape(NW, CHUNKS_PER_W, CHUNK)
    zeros = jnp.zeros((ROWS_PER_TILE, D_FEAT), jnp.float32)

    agg = _sc_agg(x, src_w, dst_w, zeros)
    agg = agg.reshape(NUM_CORES, N_PAD, D_FEAT)

    out = pl.pallas_call(
        _tc_body,
        grid=(N_BLOCKS,),
        in_specs=[
            pl.BlockSpec((NUM_CORES, ROW_BLOCK, D_FEAT), lambda i: (0, i, 0)),
            pl.BlockSpec((ROW_BLOCK, D_FEAT), lambda i: (i, 0)),
            pl.BlockSpec((D_FEAT, D_FEAT), lambda i: (0, 0)),
            pl.BlockSpec((D_FEAT, D_FEAT), lambda i: (0, 0)),
            pl.BlockSpec((1, D_FEAT), lambda i: (0, 0)),
        ],
        out_specs=pl.BlockSpec((ROW_BLOCK, D_FEAT), lambda i: (i, 0)),
        out_shape=jax.ShapeDtypeStruct((N_NODES, D_FEAT), jnp.float32),
    )(agg, x, W, W_self, b.reshape(1, D_FEAT))
    return out


# speedup vs baseline: 7.9816x; 1.4092x over previous
import functools

import jax
import jax.numpy as jnp
from jax import lax
from jax.experimental import pallas as pl
from jax.experimental.pallas import tpu as pltpu
from jax.experimental.pallas import tpu_sc as plsc

N_NODES = 10000
N_EDGES = 320000
D_FEAT = 128

NUM_CORES = 2
NUM_SUBCORES = 16
NW = NUM_CORES * NUM_SUBCORES

CHUNK = 120
ROWS_PER_TILE = 632
N_PAD = NUM_SUBCORES * ROWS_PER_TILE

NBUF = 3
NGRP = 28
CHUNKS_PER_W = NGRP * NBUF
EDGES_PER_W_PAD = CHUNKS_PER_W * CHUNK
E_PAD = EDGES_PER_W_PAD * NW

ROW_BLOCK = 2000
N_BLOCKS = N_NODES // ROW_BLOCK


def _sc_agg_body(x_hbm, sd_hbm, zeros_hbm, agg_hbm,
                 sd_v, rows_v, agg_sh, *sems):
    gsem = sems[:NBUF]
    ssem = sems[NBUF:2 * NBUF]
    isem = sems[2 * NBUF]
    c = lax.axis_index("c")
    s = lax.axis_index("s")
    w = c * NUM_SUBCORES + s

    pltpu.sync_copy(sd_hbm.at[w, 0], sd_v.at[0])
    pltpu.async_copy(sd_hbm.at[w, 1], sd_v.at[1], isem)
    for b in range(NBUF):
        pltpu.async_copy(x_hbm.at[sd_v.at[0, 0, b]], rows_v.at[b], gsem[b])
    pltpu.sync_copy(zeros_hbm, agg_sh.at[pl.ds(s * ROWS_PER_TILE, ROWS_PER_TILE)])
    plsc.subcore_barrier()

    def grp(g, carry):
        p = g & 1
        q = 1 - p
        pltpu.make_async_copy(sd_hbm.at[w, g], sd_v.at[q], isem).wait()
        for b in range(NBUF):
            pltpu.make_async_copy(
                x_hbm.at[sd_v.at[p, 0, b]], rows_v.at[b], gsem[b]).wait()
            pltpu.async_copy(
                rows_v.at[b], agg_sh.at[sd_v.at[p, 1, b]], ssem[b], add=True)
            pltpu.make_async_copy(
                rows_v.at[b], agg_sh.at[sd_v.at[p, 1, b]], ssem[b]).wait()
            pltpu.async_copy(
                x_hbm.at[sd_v.at[q, 0, b]], rows_v.at[b], gsem[b])
        gnext = jnp.minimum(g + 2, NGRP - 1)
        pltpu.async_copy(sd_hbm.at[w, gnext], sd_v.at[p], isem)
        return carry

    lax.fori_loop(0, NGRP - 1, grp, 0)

    pl_ = (NGRP - 1) & 1
    pltpu.make_async_copy(sd_hbm.at[w, 0], sd_v.at[1 - pl_], isem).wait()
    for b in range(NBUF):
        pltpu.make_async_copy(
            x_hbm.at[sd_v.at[pl_, 0, b]], rows_v.at[b], gsem[b]).wait()
        pltpu.sync_copy(rows_v.at[b], agg_sh.at[sd_v.at[pl_, 1, b]], add=True)
    plsc.subcore_barrier()

    pltpu.sync_copy(
        agg_sh.at[pl.ds(s * ROWS_PER_TILE, ROWS_PER_TILE)],
        agg_hbm.at[pl.ds(c * N_PAD + s * ROWS_PER_TILE, ROWS_PER_TILE)],
    )


_sc_agg = functools.partial(
    pl.kernel,
    out_type=jax.ShapeDtypeStruct((NUM_CORES * N_PAD, D_FEAT), jnp.float32),
    mesh=plsc.VectorSubcoreMesh(core_axis_name="c", subcore_axis_name="s"),
    scratch_types=[
        pltpu.VMEM((2, 2, NBUF, CHUNK), jnp.int32),
        pltpu.VMEM((NBUF, CHUNK, D_FEAT), jnp.float32),
        pltpu.VMEM_SHARED((N_PAD, D_FEAT), jnp.float32),
    ] + [pltpu.SemaphoreType.DMA] * (2 * NBUF + 1),
)(_sc_agg_body)


def _tc_body(agg_ref, x_ref, w_ref, ws_ref, b_ref, o_ref):
    a = agg_ref[0] + agg_ref[1]
    acc = jnp.dot(a, w_ref[...], preferred_element_type=jnp.float32)
    acc = acc + jnp.dot(x_ref[...], ws_ref[...], preferred_element_type=jnp.float32)
    acc = acc + b_ref[...]
    o_ref[...] = jnp.maximum(acc, 0.0)


@jax.jit
def kernel(x, edge_index, W, W_self, b):
    src = edge_index[0]
    dst = edge_index[1]
    pad = E_PAD - N_EDGES
    src_p = jnp.concatenate([src, jnp.zeros((pad,), jnp.int32)])
    dst_p = jnp.concatenate([dst, jnp.full((pad,), N_NODES, jnp.int32)])
    src_w = src_p.reshape(NW, NGRP, NBUF, CHUNK)
    dst_w = dst_p.reshape(NW, NGRP, NBUF, CHUNK)
    sd = jnp.stack([src_w, dst_w], axis=2)
    zeros = jnp.zeros((ROWS_PER_TILE, D_FEAT), jnp.float32)

    agg = _sc_agg(x, sd, zeros)
    agg = agg.reshape(NUM_CORES, N_PAD, D_FEAT)

    out = pl.pallas_call(
        _tc_body,
        grid=(N_BLOCKS,),
        in_specs=[
            pl.BlockSpec((NUM_CORES, ROW_BLOCK, D_FEAT), lambda i: (0, i, 0)),
            pl.BlockSpec((ROW_BLOCK, D_FEAT), lambda i: (i, 0)),
            pl.BlockSpec((D_FEAT, D_FEAT), lambda i: (0, 0)),
            pl.BlockSpec((D_FEAT, D_FEAT), lambda i: (0, 0)),
            pl.BlockSpec((1, D_FEAT), lambda i: (0, 0)),
        ],
        out_specs=pl.BlockSpec((ROW_BLOCK, D_FEAT), lambda i: (i, 0)),
        out_shape=jax.ShapeDtypeStruct((N_NODES, D_FEAT), jnp.float32),
    )(agg, x, W, W_self, b.reshape(1, D_FEAT))
    return out
